# baseline (speedup 1.0000x reference)
import jax
import jax.numpy as jnp
from jax import lax
from jax.experimental import pallas as pl
from jax.experimental.pallas import tpu as pltpu
from jax.experimental.pallas import tpu_sc as plsc

_NC = 2
_NS = 16
_NW = _NC * _NS

_B, _S = 16384, 50
_TOTAL = _B * _S
_BLK = 128
_NBLK = _TOTAL // _BLK
_PER_W = _NBLK // _NW
_D = 64
_VROWS = 500000

_NBUF = 3


def _body(idx_hbm, tbl_hbm, out_hbm, idx_v, pidx_v, colb_v, gath_v, rowsT_v,
          gsem, osem):
    wid = lax.axis_index("s") * _NC + lax.axis_index("c")
    base = wid * _PER_W
    pltpu.sync_copy(idx_hbm.at[pl.ds(base, _PER_W)], idx_v)

    lane = lax.iota(jnp.int32, 16)

    def prep_indices(t, slot):
        for k in range(8):
            v = idx_v[t, pl.ds(k * 16, 16)]
            pidx_v[slot, pl.ds(k * 16, 16)] = lax.shift_right_logical(v, 1)
            colb_v[slot, pl.ds(k * 16, 16)] = lax.shift_left(
                lax.bitwise_and(v, 1), 6)

    def start_gather(slot):
        pltpu.async_copy(tbl_hbm.at[pidx_v.at[slot]], gath_v.at[slot],
                         gsem.at[slot])

    def wait_gather(slot):
        pltpu.make_async_copy(tbl_hbm.at[pidx_v.at[slot]], gath_v.at[slot],
                              gsem.at[slot]).wait()

    def select(slot):
        def dstep(d, carry):
            for j0 in range(8):
                rows = lane + (j0 * 16)
                cols = colb_v[slot, pl.ds(j0 * 16, 16)] + d
                vals = plsc.load_gather(gath_v.at[slot], [rows, cols])
                rowsT_v[slot, d, pl.ds(j0 * 16, 16)] = vals
            return carry
        lax.fori_loop(0, _D, dstep, 0)

    def start_out(t, slot):
        g = base + t
        s = lax.shift_right_logical(g, 7)
        c = lax.bitwise_and(g, 127)
        pltpu.async_copy(rowsT_v.at[slot],
                         out_hbm.at[s, :, pl.ds(c * _BLK, _BLK)],
                         osem.at[slot])

    def wait_out(slot):
        pltpu.make_async_copy(rowsT_v.at[slot],
                              out_hbm.at[0, :, pl.ds(0, _BLK)],
                              osem.at[slot]).wait()

    for u in range(_NBUF):
        prep_indices(u, u)
        start_gather(u)

    def step(t, slot, first, last):
        wait_gather(slot)
        if not first:
            wait_out(slot)
        select(slot)
        start_out(t, slot)
        if not last:
            prep_indices(t + _NBUF, slot)
            start_gather(slot)

    for t in range(_NBUF):
        step(t, t, True, False)

    n_main = (_PER_W - 2 * _NBUF) // _NBUF

    def outer(o, carry):
        t0 = _NBUF + o * _NBUF
        for j in range(_NBUF):
            step(t0 + j, j, False, False)
        return carry

    lax.fori_loop(0, n_main, outer, 0)

    for t in range(_NBUF + n_main * _NBUF, _PER_W):
        step(t, t % _NBUF, False, t + _NBUF >= _PER_W)
    for slot in range(_NBUF):
        wait_out(slot)


_sc_gather = pl.kernel(
    _body,
    out_type=jax.ShapeDtypeStruct((_S, _D, _B), jnp.float32),
    mesh=plsc.VectorSubcoreMesh(core_axis_name="c", subcore_axis_name="s"),
    scratch_types=[
        pltpu.VMEM((_PER_W, _BLK), jnp.int32),
        pltpu.VMEM((_NBUF, _BLK), jnp.int32),
        pltpu.VMEM((_NBUF, _BLK), jnp.int32),
        pltpu.VMEM((_NBUF, _BLK, 128), jnp.float32),
        pltpu.VMEM((_NBUF, _D, _BLK), jnp.float32),
        pltpu.SemaphoreType.DMA((_NBUF,)),
        pltpu.SemaphoreType.DMA((_NBUF,)),
    ],
    compiler_params=pltpu.CompilerParams(needs_layout_passes=False),
)


def kernel(token_ids, embedding_matrix):
    idx = token_ids.T.reshape(_NBLK, _BLK).astype(jnp.int32)
    tbl2 = embedding_matrix.reshape(_VROWS, 2 * _D)
    out = _sc_gather(idx, tbl2)
    return jnp.transpose(out, (2, 0, 1))


# speedup vs baseline: 1.3153x; 1.3153x over previous
import jax
import jax.numpy as jnp
from jax import lax
from jax.experimental import pallas as pl
from jax.experimental.pallas import tpu as pltpu
from jax.experimental.pallas import tpu_sc as plsc

_NC = 2
_NS = 16
_NW = _NC * _NS

_B, _S = 16384, 50
_TOTAL = _B * _S
_BLK = 128
_NBLK = _TOTAL // _BLK
_PER_W = _NBLK // _NW
_D = 64
_VROWS = 500000

_NBUF = 3


def _body(idx_hbm, tbl_hbm, out_hbm, idx_v, pidx_v, colb_v, gath_v, rowsT_v,
          gsem, osem):
    wid = lax.axis_index("s") * _NC + lax.axis_index("c")
    base = wid * _PER_W
    pltpu.sync_copy(idx_hbm.at[pl.ds(base, _PER_W)], idx_v)

    lane = lax.iota(jnp.int32, 16)

    def prep_indices(t, slot):
        for k in range(8):
            v = idx_v[t, pl.ds(k * 16, 16)]
            pidx_v[slot, pl.ds(k * 16, 16)] = lax.shift_right_logical(v, 1)
            colb_v[slot, pl.ds(k * 16, 16)] = lax.shift_left(
                lax.bitwise_and(v, 1), 6)

    def start_gather(slot):
        pltpu.async_copy(tbl_hbm.at[pidx_v.at[slot]], gath_v.at[slot],
                         gsem.at[slot])

    def wait_gather(slot):
        pltpu.make_async_copy(tbl_hbm.at[pidx_v.at[slot]], gath_v.at[slot],
                              gsem.at[slot]).wait()

    def select(slot):
        def dstep(d8, carry):
            d0 = d8 * 8
            for j0 in range(8):
                rows = lane + (j0 * 16)
                cols = colb_v[slot, pl.ds(j0 * 16, 16)]
                for k in range(8):
                    vals = plsc.load_gather(gath_v.at[slot],
                                            [rows, cols + (d0 + k)])
                    rowsT_v[slot, d0 + k, pl.ds(j0 * 16, 16)] = vals
            return carry
        lax.fori_loop(0, 8, dstep, 0)

    def start_out(t, slot):
        g = base + t
        s = lax.shift_right_logical(g, 7)
        c = lax.bitwise_and(g, 127)
        pltpu.async_copy(rowsT_v.at[slot],
                         out_hbm.at[s, :, pl.ds(c * _BLK, _BLK)],
                         osem.at[slot])

    def wait_out(slot):
        pltpu.make_async_copy(rowsT_v.at[slot],
                              out_hbm.at[0, :, pl.ds(0, _BLK)],
                              osem.at[slot]).wait()

    for u in range(_NBUF):
        prep_indices(u, u)
        start_gather(u)

    def step(t, slot, first, last):
        wait_gather(slot)
        if not first:
            wait_out(slot)
        select(slot)
        start_out(t, slot)
        if not last:
            prep_indices(t + _NBUF, slot)
            start_gather(slot)

    for t in range(_NBUF):
        step(t, t, True, False)

    n_main = (_PER_W - 2 * _NBUF) // _NBUF

    def outer(o, carry):
        t0 = _NBUF + o * _NBUF
        for j in range(_NBUF):
            step(t0 + j, j, False, False)
        return carry

    lax.fori_loop(0, n_main, outer, 0)

    for t in range(_NBUF + n_main * _NBUF, _PER_W):
        step(t, t % _NBUF, False, t + _NBUF >= _PER_W)
    for slot in range(_NBUF):
        wait_out(slot)


_sc_gather = pl.kernel(
    _body,
    out_type=jax.ShapeDtypeStruct((_S, _D, _B), jnp.float32),
    mesh=plsc.VectorSubcoreMesh(core_axis_name="c", subcore_axis_name="s"),
    scratch_types=[
        pltpu.VMEM((_PER_W, _BLK), jnp.int32),
        pltpu.VMEM((_NBUF, _BLK), jnp.int32),
        pltpu.VMEM((_NBUF, _BLK), jnp.int32),
        pltpu.VMEM((_NBUF, _BLK, 128), jnp.float32),
        pltpu.VMEM((_NBUF, _D, _BLK), jnp.float32),
        pltpu.SemaphoreType.DMA((_NBUF,)),
        pltpu.SemaphoreType.DMA((_NBUF,)),
    ],
    compiler_params=pltpu.CompilerParams(needs_layout_passes=False),
)


def kernel(token_ids, embedding_matrix):
    idx = token_ids.T.reshape(_NBLK, _BLK).astype(jnp.int32)
    tbl2 = embedding_matrix.reshape(_VROWS, 2 * _D)
    out = _sc_gather(idx, tbl2)
    return jnp.transpose(out, (2, 0, 1))
